# baseline (speedup 1.0000x reference)
import functools
import itertools

import jax
import jax.numpy as jnp
from jax import lax
from jax.experimental import pallas as pl
from jax.experimental.pallas import tpu as pltpu
from jax.experimental.pallas import tpu_sc as plsc

_WIDTHS = tuple([3] * 16 + [5] * 16 + [7] * 16)
_SEG_STARTS = tuple(int(x) for x in itertools.accumulate((0,) + _WIDTHS[:-1]))
_NUM_SEG = 48
_M = 240
_CR = 160
_NW = 32


def _sc_body(x_hbm, cgseg_hbm, out_hbm, xbuf, obuf, cgbuf, sem_in, sem_out):
    c = lax.axis_index("c")
    s = lax.axis_index("s")
    wid = s * 2 + c
    pltpu.sync_copy(cgseg_hbm, cgbuf)
    cgv = [cgbuf[pl.ds(16 * j, 16)] for j in range(_NUM_SEG // 16)]

    n_chunks_total = x_hbm.shape[0] // _CR
    n_mine = (n_chunks_total - wid + _NW - 1) // _NW
    rows16 = lax.iota(jnp.int32, 16)

    def chunk_body(ci, carry):
        chunk = wid + ci * _NW
        cp_in = pltpu.make_async_copy(
            x_hbm.at[pl.ds(chunk * _CR, _CR), :], xbuf, sem_in
        )
        cp_in.start()
        cp_in.wait()

        def group_body(gl, gcarry):
            rows = rows16 + gl * 16
            for n in range(_NUM_SEG):
                acc = None
                for k in range(_WIDTHS[n]):
                    m = _SEG_STARTS[n] + k
                    cols = jnp.full((16,), m, dtype=jnp.int32)
                    v = plsc.load_gather(xbuf, [rows, cols])
                    t = v * v
                    acc = t if acc is None else acc + t
                accw = acc * cgv[n // 16][n % 16]
                ncols = jnp.full((16,), n, dtype=jnp.int32)
                plsc.store_scatter(obuf, [rows, ncols], accw)
            return gcarry

        lax.fori_loop(0, _CR // 16, group_body, 0)
        cp_out = pltpu.make_async_copy(
            obuf, out_hbm.at[pl.ds(chunk * _CR, _CR), :], sem_out
        )
        cp_out.start()
        cp_out.wait()
        return carry

    lax.fori_loop(0, n_mine, chunk_body, 0)


def kernel(sphc, cg_rep, segment_ids):
    B, M = sphc.shape
    Bp = pl.cdiv(B, _CR) * _CR
    if Bp != B:
        sphc = jnp.pad(sphc, ((0, Bp - B), (0, 0)))
    cg_seg = cg_rep[jnp.array(_SEG_STARTS, dtype=jnp.int32)]
    run = functools.partial(
        pl.kernel,
        out_type=jax.ShapeDtypeStruct((Bp, _NUM_SEG), jnp.float32),
        mesh=plsc.VectorSubcoreMesh(core_axis_name="c", subcore_axis_name="s"),
        scratch_types=[
            pltpu.VMEM((_CR, _M), jnp.float32),
            pltpu.VMEM((_CR, _NUM_SEG), jnp.float32),
            pltpu.VMEM((_NUM_SEG,), jnp.float32),
            pltpu.SemaphoreType.DMA,
            pltpu.SemaphoreType.DMA,
        ],
        compiler_params=pltpu.CompilerParams(use_tc_tiling_on_sc=False, needs_layout_passes=False),
    )(_sc_body)
    out = run(sphc, cg_seg)
    return out[:B] if Bp != B else out


# speedup vs baseline: 1.0502x; 1.0502x over previous
import functools
import itertools

import jax
import jax.numpy as jnp
from jax import lax
from jax.experimental import pallas as pl
from jax.experimental.pallas import tpu as pltpu
from jax.experimental.pallas import tpu_sc as plsc

_WIDTHS = tuple([3] * 16 + [5] * 16 + [7] * 16)
_SEG_STARTS = tuple(int(x) for x in itertools.accumulate((0,) + _WIDTHS[:-1]))
_NUM_SEG = 48
_M = 240
_CR = 160
_NW = 32


def _sc_body(x_hbm, cgseg_hbm, out_hbm, xbuf, obuf, cgbuf, sem_in, sem_out):
    c = lax.axis_index("c")
    s = lax.axis_index("s")
    wid = s * 2 + c
    pltpu.sync_copy(cgseg_hbm, cgbuf)
    cgv = [cgbuf[pl.ds(16 * j, 16)] for j in range(_NUM_SEG // 16)]

    n_chunks_total = x_hbm.shape[0] // _CR
    n_mine = (n_chunks_total - wid + _NW - 1) // _NW
    rows16 = lax.iota(jnp.int32, 16)

    def chunk_body(ci, carry):
        chunk = wid + ci * _NW
        cp_in = pltpu.make_async_copy(
            x_hbm.at[pl.ds(chunk * _CR, _CR), :], xbuf.at[:, pl.ds(0, _M)], sem_in
        )
        cp_in.start()
        cp_in.wait()

        def group_body(gl, gcarry):
            rows = rows16 + gl * 16
            for n in range(_NUM_SEG):
                acc = None
                for k in range(_WIDTHS[n]):
                    m = _SEG_STARTS[n] + k
                    cols = jnp.full((16,), m, dtype=jnp.int32)
                    v = plsc.load_gather(xbuf, [rows, cols])
                    t = v * v
                    acc = t if acc is None else acc + t
                accw = acc * cgv[n // 16][n % 16]
                ncols = jnp.full((16,), n, dtype=jnp.int32)
                plsc.store_scatter(obuf, [rows, ncols], accw)
            return gcarry

        lax.fori_loop(0, _CR // 16, group_body, 0)
        cp_out = pltpu.make_async_copy(
            obuf.at[:, pl.ds(0, _NUM_SEG)], out_hbm.at[pl.ds(chunk * _CR, _CR), :], sem_out
        )
        cp_out.start()
        cp_out.wait()
        return carry

    lax.fori_loop(0, n_mine, chunk_body, 0)


def kernel(sphc, cg_rep, segment_ids):
    B, M = sphc.shape
    Bp = pl.cdiv(B, _CR) * _CR
    if Bp != B:
        sphc = jnp.pad(sphc, ((0, Bp - B), (0, 0)))
    cg_seg = cg_rep[jnp.array(_SEG_STARTS, dtype=jnp.int32)]
    run = functools.partial(
        pl.kernel,
        out_type=jax.ShapeDtypeStruct((Bp, _NUM_SEG), jnp.float32),
        mesh=plsc.VectorSubcoreMesh(core_axis_name="c", subcore_axis_name="s"),
        scratch_types=[
            pltpu.VMEM((_CR, _M + 1), jnp.float32),
            pltpu.VMEM((_CR, _NUM_SEG + 1), jnp.float32),
            pltpu.VMEM((_NUM_SEG,), jnp.float32),
            pltpu.SemaphoreType.DMA,
            pltpu.SemaphoreType.DMA,
        ],
        compiler_params=pltpu.CompilerParams(use_tc_tiling_on_sc=False, needs_layout_passes=False),
    )(_sc_body)
    out = run(sphc, cg_seg)
    return out[:B] if Bp != B else out


# speedup vs baseline: 1.3214x; 1.2583x over previous
import functools
import itertools

import jax
import jax.numpy as jnp
from jax import lax
from jax.experimental import pallas as pl
from jax.experimental.pallas import tpu as pltpu
from jax.experimental.pallas import tpu_sc as plsc

_WIDTHS = tuple([3] * 16 + [5] * 16 + [7] * 16)
_SEG_STARTS = tuple(int(x) for x in itertools.accumulate((0,) + _WIDTHS[:-1]))
_NUM_SEG = 48
_M = 240
_CR = 160
_NW = 32


def _sc_body(x_hbm, cgseg_hbm, out_hbm, xbuf, obuf, cgbuf, sem_in, sem_out):
    c = lax.axis_index("c")
    s = lax.axis_index("s")
    wid = s * 2 + c
    pltpu.sync_copy(cgseg_hbm, cgbuf)
    cgv = [cgbuf[pl.ds(16 * j, 16)] for j in range(_NUM_SEG // 16)]

    n_chunks_total = x_hbm.shape[0] // _CR
    n_mine = (n_chunks_total - wid + _NW - 1) // _NW
    rows16 = lax.iota(jnp.int32, 16)

    def chunk_body(ci, carry):
        chunk = wid + ci * _NW
        cp_in = pltpu.make_async_copy(
            x_hbm.at[pl.ds(chunk * _CR, _CR), :], xbuf, sem_in
        )
        cp_in.start()
        cp_in.wait()

        def group_body(gl, gcarry):
            rows = rows16 + gl * 16
            for n in range(_NUM_SEG):
                acc = None
                for k in range(_WIDTHS[n]):
                    m = _SEG_STARTS[n] + k
                    cols = jnp.full((16,), m, dtype=jnp.int32)
                    v = plsc.load_gather(xbuf, [rows, cols])
                    t = v * v
                    acc = t if acc is None else acc + t
                accw = acc * cgv[n // 16][n % 16]
                ncols = jnp.full((16,), n, dtype=jnp.int32)
                plsc.store_scatter(obuf, [rows, ncols], accw)
            return gcarry

        lax.fori_loop(0, _CR // 16, group_body, 0)
        cp_out = pltpu.make_async_copy(
            obuf, out_hbm.at[pl.ds(chunk * _CR, _CR), :], sem_out
        )
        cp_out.start()
        cp_out.wait()
        return carry

    lax.fori_loop(0, n_mine, chunk_body, 0)


def kernel(sphc, cg_rep, segment_ids):
    B, M = sphc.shape
    Bp = pl.cdiv(B, _CR) * _CR
    if Bp != B:
        sphc = jnp.pad(sphc, ((0, Bp - B), (0, 0)))
    cg_seg = cg_rep[jnp.array(_SEG_STARTS, dtype=jnp.int32)]
    run = functools.partial(
        pl.kernel,
        out_type=jax.ShapeDtypeStruct((Bp, _NUM_SEG), jnp.float32),
        mesh=plsc.VectorSubcoreMesh(core_axis_name="c", subcore_axis_name="s"),
        scratch_types=[
            pltpu.VMEM((_CR, _M), jnp.float32),
            pltpu.VMEM((_CR, _NUM_SEG), jnp.float32),
            pltpu.VMEM((_NUM_SEG,), jnp.float32),
            pltpu.SemaphoreType.DMA,
            pltpu.SemaphoreType.DMA,
        ],
        compiler_params=pltpu.CompilerParams(use_tc_tiling_on_sc=True, needs_layout_passes=False),
    )(_sc_body)
    out = run(sphc, cg_seg)
    return out[:B] if Bp != B else out


# speedup vs baseline: 1.3217x; 1.0002x over previous
import functools
import itertools

import jax
import jax.numpy as jnp
from jax import lax
from jax.experimental import pallas as pl
from jax.experimental.pallas import tpu as pltpu
from jax.experimental.pallas import tpu_sc as plsc

_WIDTHS = tuple([3] * 16 + [5] * 16 + [7] * 16)
_SEG_STARTS = tuple(int(x) for x in itertools.accumulate((0,) + _WIDTHS[:-1]))
_NUM_SEG = 48
_M = 240
_CR = 160
_NW = 32


def _tree_sum(terms):
    while len(terms) > 1:
        nxt = [a + b for a, b in zip(terms[::2], terms[1::2])]
        if len(terms) % 2:
            nxt.append(terms[-1])
        terms = nxt
    return terms[0]


def _sc_body(x_hbm, cgseg_hbm, out_hbm, xb0, xb1, obuf, cgbuf, si0, si1, so):
    c = lax.axis_index("c")
    s = lax.axis_index("s")
    wid = s * 2 + c
    pltpu.sync_copy(cgseg_hbm, cgbuf)
    n_chunks_total = x_hbm.shape[0] // _CR
    n_mine = (n_chunks_total - wid + _NW - 1) // _NW
    rows16 = lax.iota(jnp.int32, 16)
    xbufs = (xb0, xb1)
    sems = (si0, si1)

    def start_in(ci, buf, sem):
        pltpu.make_async_copy(
            x_hbm.at[pl.ds((wid + ci * _NW) * _CR, _CR), :], buf, sem
        ).start()

    @pl.when(n_mine > 0)
    def _prime():
        start_in(0, xb0, si0)

    def compute_chunk(ci, xbuf):
        def group_body(gl, cols0):
            rows = rows16 + gl * 16
            cols = cols0
            ncols = cols0
            for n in range(_NUM_SEG):
                terms = []
                for _ in range(_WIDTHS[n]):
                    v = plsc.load_gather(xbuf, [rows, cols])
                    cols = cols + 1
                    terms.append(v * v)
                acc = _tree_sum(terms)
                cg = plsc.load_gather(cgbuf, [ncols])
                plsc.store_scatter(obuf, [rows, ncols], acc * cg)
                ncols = ncols + 1
            return cols0

        lax.fori_loop(0, _CR // 16, group_body, jnp.zeros((16,), jnp.int32))
        cp_out = pltpu.make_async_copy(
            obuf, out_hbm.at[pl.ds((wid + ci * _NW) * _CR, _CR), :], so
        )
        cp_out.start()
        cp_out.wait()

    def pair_body(pi, carry):
        for b in range(2):
            ci = pi * 2 + b
            nxt, nbuf, nsem = ci + 1, xbufs[1 - b], sems[1 - b]

            @pl.when(nxt < n_mine)
            def _start_next():
                start_in(nxt, nbuf, nsem)

            @pl.when(ci < n_mine)
            def _do():
                pltpu.make_async_copy(
                    x_hbm.at[pl.ds((wid + ci * _NW) * _CR, _CR), :],
                    xbufs[b],
                    sems[b],
                ).wait()
                compute_chunk(ci, xbufs[b])
        return carry

    lax.fori_loop(0, (n_mine + 1) // 2, pair_body, 0)


def kernel(sphc, cg_rep, segment_ids):
    B, M = sphc.shape
    Bp = pl.cdiv(B, _CR) * _CR
    if Bp != B:
        sphc = jnp.pad(sphc, ((0, Bp - B), (0, 0)))
    cg_seg = cg_rep[jnp.array(_SEG_STARTS, dtype=jnp.int32)]
    run = functools.partial(
        pl.kernel,
        out_type=jax.ShapeDtypeStruct((Bp, _NUM_SEG), jnp.float32),
        mesh=plsc.VectorSubcoreMesh(core_axis_name="c", subcore_axis_name="s"),
        scratch_types=[
            pltpu.VMEM((_CR, _M), jnp.float32),
            pltpu.VMEM((_CR, _M), jnp.float32),
            pltpu.VMEM((_CR, _NUM_SEG), jnp.float32),
            pltpu.VMEM((_NUM_SEG,), jnp.float32),
            pltpu.SemaphoreType.DMA,
            pltpu.SemaphoreType.DMA,
            pltpu.SemaphoreType.DMA,
        ],
        compiler_params=pltpu.CompilerParams(
            use_tc_tiling_on_sc=True, needs_layout_passes=False
        ),
    )(_sc_body)
    out = run(sphc, cg_seg)
    return out[:B] if Bp != B else out


# speedup vs baseline: 3.7035x; 2.8020x over previous
import functools
import itertools

import jax
import jax.numpy as jnp
from jax import lax
from jax.experimental import pallas as pl
from jax.experimental.pallas import tpu as pltpu
from jax.experimental.pallas import tpu_sc as plsc

_WIDTHS = tuple([3] * 16 + [5] * 16 + [7] * 16)
_SEG_STARTS = tuple(int(x) for x in itertools.accumulate((0,) + _WIDTHS[:-1]))
_NUM_SEG = 48
_M = 240
_CR = 160
_NW = 32


def _tree_sum(terms):
    while len(terms) > 1:
        nxt = [a + b for a, b in zip(terms[::2], terms[1::2])]
        if len(terms) % 2:
            nxt.append(terms[-1])
        terms = nxt
    return terms[0]


_K_SC = 9600


def _tc_body(x_ref, s_ref, o_ref):
    x = x_ref[...]
    o_ref[...] = jnp.dot(x * x, s_ref[...], preferred_element_type=jnp.float32)


def _sc_body(x_hbm, cgseg_hbm, out_hbm, xb0, xb1, obuf, cgbuf, si0, si1, so):
    c = lax.axis_index("c")
    s = lax.axis_index("s")
    wid = s * 2 + c
    pltpu.sync_copy(cgseg_hbm, cgbuf)
    n_chunks_total = _K_SC // _CR
    n_mine = (n_chunks_total - wid + _NW - 1) // _NW
    rows16 = lax.iota(jnp.int32, 16)
    xbufs = (xb0, xb1)
    sems = (si0, si1)

    def start_in(ci, buf, sem):
        pltpu.make_async_copy(
            x_hbm.at[pl.ds((wid + ci * _NW) * _CR, _CR), :], buf, sem
        ).start()

    @pl.when(n_mine > 0)
    def _prime():
        start_in(0, xb0, si0)

    def compute_chunk(ci, xbuf):
        def group_body(gl, cols0):
            rows = rows16 + gl * 16
            cols = cols0
            ncols = cols0
            for n in range(_NUM_SEG):
                terms = []
                for _ in range(_WIDTHS[n]):
                    v = plsc.load_gather(xbuf, [rows, cols])
                    cols = cols + 1
                    terms.append(v * v)
                acc = _tree_sum(terms)
                cg = plsc.load_gather(cgbuf, [ncols])
                plsc.store_scatter(obuf, [rows, ncols], acc * cg)
                ncols = ncols + 1
            return cols0

        lax.fori_loop(0, _CR // 16, group_body, jnp.zeros((16,), jnp.int32))
        cp_out = pltpu.make_async_copy(
            obuf, out_hbm.at[pl.ds((wid + ci * _NW) * _CR, _CR), :], so
        )
        cp_out.start()
        cp_out.wait()

    def pair_body(pi, carry):
        for b in range(2):
            ci = pi * 2 + b
            nxt, nbuf, nsem = ci + 1, xbufs[1 - b], sems[1 - b]

            @pl.when(nxt < n_mine)
            def _start_next():
                start_in(nxt, nbuf, nsem)

            @pl.when(ci < n_mine)
            def _do():
                pltpu.make_async_copy(
                    x_hbm.at[pl.ds((wid + ci * _NW) * _CR, _CR), :],
                    xbufs[b],
                    sems[b],
                ).wait()
                compute_chunk(ci, xbufs[b])
        return carry

    lax.fori_loop(0, (n_mine + 1) // 2, pair_body, 0)


def kernel(sphc, cg_rep, segment_ids):
    B, M = sphc.shape
    Bp = pl.cdiv(B, _CR) * _CR
    if Bp != B:
        sphc = jnp.pad(sphc, ((0, Bp - B), (0, 0)))
    cg_seg = cg_rep[jnp.array(_SEG_STARTS, dtype=jnp.int32)]
    run = functools.partial(
        pl.kernel,
        out_type=jax.ShapeDtypeStruct((_K_SC, _NUM_SEG), jnp.float32),
        mesh=plsc.VectorSubcoreMesh(core_axis_name="c", subcore_axis_name="s"),
        scratch_types=[
            pltpu.VMEM((_CR, _M), jnp.float32),
            pltpu.VMEM((_CR, _M), jnp.float32),
            pltpu.VMEM((_CR, _NUM_SEG), jnp.float32),
            pltpu.VMEM((_NUM_SEG,), jnp.float32),
            pltpu.SemaphoreType.DMA,
            pltpu.SemaphoreType.DMA,
            pltpu.SemaphoreType.DMA,
        ],
        compiler_params=pltpu.CompilerParams(
            use_tc_tiling_on_sc=True, needs_layout_passes=False
        ),
    )(_sc_body)
    sc_out = run(sphc, cg_seg)[:_K_SC]
    seg = segment_ids.astype(jnp.int32)
    proj = (
        seg[:, None] == jnp.arange(_NUM_SEG, dtype=jnp.int32)[None, :]
    ).astype(jnp.float32) * cg_rep[:, None]
    TB = 800
    k_blocks = _K_SC // TB
    grid = ((B - _K_SC) // TB,)
    tc_out = pl.pallas_call(
        _tc_body,
        grid=grid,
        in_specs=[
            pl.BlockSpec((TB, M), lambda i: (i + k_blocks, 0)),
            pl.BlockSpec((M, _NUM_SEG), lambda i: (0, 0)),
        ],
        out_specs=pl.BlockSpec((TB, _NUM_SEG), lambda i: (i + k_blocks, 0)),
        out_shape=jax.ShapeDtypeStruct((B, _NUM_SEG), jnp.float32),
        compiler_params=pltpu.CompilerParams(
            dimension_semantics=("arbitrary",),
        ),
    )(sphc, proj)
    return lax.dynamic_update_slice(tc_out, sc_out, (0, 0))
